# baseline (speedup 1.0000x reference)
import functools

import jax
import jax.numpy as jnp
from jax import lax
from jax.experimental import pallas as pl
from jax.experimental.pallas import tpu as pltpu
from jax.experimental.pallas import tpu_sc as plsc

EMB = 32
GRP = 128 // EMB
NC, NS = 2, 16
NW = NC * NS
CHUNK = 128


def _sc_gather_make(batch):
    bpw = batch // NW
    cpw = bpw // CHUNK

    @functools.partial(
        pl.kernel,
        out_type=(
            jax.ShapeDtypeStruct((batch, 128), jnp.float32),
            jax.ShapeDtypeStruct((batch, 128), jnp.float32),
        ),
        mesh=plsc.VectorSubcoreMesh(core_axis_name="c", subcore_axis_name="s"),
        scratch_types=[
            pltpu.VMEM((cpw, CHUNK), jnp.int32),
            pltpu.VMEM((cpw, CHUNK), jnp.int32),
            pltpu.VMEM((bpw, 128), jnp.float32),
            pltpu.SemaphoreType.DMA,
        ],
    )
    def sc_gather(uhi_hbm, ihi_hbm, utab_hbm, itab_hbm,
                  uout_hbm, iout_hbm, uhi_v, ihi_v, rows_v, sem):
        wid = lax.axis_index("s") * NC + lax.axis_index("c")
        base = wid * bpw
        pltpu.sync_copy(uhi_hbm.at[pl.ds(wid * cpw, cpw)], uhi_v)
        pltpu.sync_copy(ihi_hbm.at[pl.ds(wid * cpw, cpw)], ihi_v)

        def gather_table(hi_v, tab_hbm, out_hbm):
            copies = []
            for c in range(cpw):
                copies.append(pltpu.async_copy(
                    tab_hbm.at[hi_v.at[c]],
                    rows_v.at[pl.ds(c * CHUNK, CHUNK)], sem))
            for cp in copies:
                cp.wait()
            pltpu.sync_copy(rows_v, out_hbm.at[pl.ds(base, bpw)])

        gather_table(uhi_v, utab_hbm, uout_hbm)
        gather_table(ihi_v, itab_hbm, iout_hbm)

    return sc_gather


def _mlp_body(u_ref, i_ref, ulo_ref, ilo_ref, w1u_ref, w1i_ref, b1_ref,
              w2_ref, b2_ref, w3_ref, b3_ref, o_ref):
    ulo = ulo_ref[...]
    ilo = ilo_ref[...]
    ug = u_ref[...]
    ig = i_ref[...]
    u = jnp.zeros(ug[:, :EMB].shape, jnp.float32)
    i = jnp.zeros_like(u)
    for g in range(GRP):
        u = u + jnp.where(ulo == g, ug[:, g * EMB:(g + 1) * EMB], 0.0)
        i = i + jnp.where(ilo == g, ig[:, g * EMB:(g + 1) * EMB], 0.0)
    h1 = jnp.dot(u, w1u_ref[...], preferred_element_type=jnp.float32)
    h1 = h1 + jnp.dot(i, w1i_ref[...], preferred_element_type=jnp.float32)
    h1 = jnp.maximum(h1 + b1_ref[...], 0.0)
    h2 = jnp.dot(h1, w2_ref[...], preferred_element_type=jnp.float32)
    h2 = jnp.maximum(h2 + b2_ref[...], 0.0)
    z = jnp.dot(h2, w3_ref[...], preferred_element_type=jnp.float32)
    o_ref[...] = jax.nn.sigmoid(z + b3_ref[...])


def kernel(user_input, item_input, user_table, item_table,
           W1, b1, W2, b2, W3, b3):
    batch = user_input.shape[0]
    cpw = batch // (NW * CHUNK)
    uidx = user_input.astype(jnp.int32)
    iidx = item_input.astype(jnp.int32)
    uhi = (uidx >> 2).reshape(NW * cpw, CHUNK)
    ihi = (iidx >> 2).reshape(NW * cpw, CHUNK)
    ulo = (uidx & (GRP - 1)).reshape(batch, 1)
    ilo = (iidx & (GRP - 1)).reshape(batch, 1)
    one = lax.optimization_barrier(jnp.ones((), jnp.float32))
    utabc = user_table.reshape(user_table.shape[0] // GRP, 128) * one
    itabc = item_table.reshape(item_table.shape[0] // GRP, 128) * one

    u_grp, i_grp = _sc_gather_make(batch)(uhi, ihi, utabc, itabc)

    bm = 2048
    pred = pl.pallas_call(
        _mlp_body,
        grid=(batch // bm,),
        in_specs=[
            pl.BlockSpec((bm, 128), lambda b: (b, 0)),
            pl.BlockSpec((bm, 128), lambda b: (b, 0)),
            pl.BlockSpec((bm, 1), lambda b: (b, 0)),
            pl.BlockSpec((bm, 1), lambda b: (b, 0)),
            pl.BlockSpec((EMB, 64), lambda b: (0, 0)),
            pl.BlockSpec((EMB, 64), lambda b: (0, 0)),
            pl.BlockSpec((1, 64), lambda b: (0, 0)),
            pl.BlockSpec((64, EMB), lambda b: (0, 0)),
            pl.BlockSpec((1, EMB), lambda b: (0, 0)),
            pl.BlockSpec((EMB, 1), lambda b: (0, 0)),
            pl.BlockSpec((1, 1), lambda b: (0, 0)),
        ],
        out_specs=pl.BlockSpec((bm, 1), lambda b: (b, 0)),
        out_shape=jax.ShapeDtypeStruct((batch, 1), jnp.float32),
    )(u_grp, i_grp, ulo, ilo, W1[:EMB], W1[EMB:], b1.reshape(1, 64),
      W2, b2.reshape(1, EMB), W3, b3.reshape(1, 1))
    return pred


# speedup vs baseline: 5.2333x; 5.2333x over previous
import functools

import jax
import jax.numpy as jnp
from jax import lax
from jax.experimental import pallas as pl
from jax.experimental.pallas import tpu as pltpu
from jax.experimental.pallas import tpu_sc as plsc

EMB = 32
NC, NS = 2, 16
NW = NC * NS
LANES = 16
NBUF = 16


def _sc_gather_make(batch):
    bpw = batch // NW

    @functools.partial(
        pl.kernel,
        out_type=(
            jax.ShapeDtypeStruct((EMB, batch), jnp.float32),
            jax.ShapeDtypeStruct((EMB, batch), jnp.float32),
        ),
        mesh=plsc.VectorSubcoreMesh(core_axis_name="c", subcore_axis_name="s"),
        scratch_types=[
            pltpu.VMEM((bpw,), jnp.int32),
            pltpu.VMEM((bpw,), jnp.int32),
            pltpu.VMEM((bpw,), jnp.int32),
            pltpu.VMEM((bpw,), jnp.int32),
            pltpu.VMEM((NBUF, EMB, 128), jnp.float32),
            pltpu.VMEM((EMB, bpw), jnp.float32),
            pltpu.VMEM((EMB, bpw), jnp.float32),
        ] + [pltpu.SemaphoreType.DMA] * NBUF,
        compiler_params=pltpu.CompilerParams(needs_layout_passes=False),
    )
    def sc_gather(uhi_hbm, ulo_hbm, ihi_hbm, ilo_hbm, utab_hbm, itab_hbm,
                  uout_hbm, iout_hbm, uhi_v, ulo_v, ihi_v, ilo_v,
                  ring_v, urows_v, irows_v, *sems):
        wid = lax.axis_index("s") * NC + lax.axis_index("c")
        base = wid * bpw
        pltpu.sync_copy(uhi_hbm.at[pl.ds(base, bpw)], uhi_v)
        pltpu.sync_copy(ulo_hbm.at[pl.ds(base, bpw)], ulo_v)
        pltpu.sync_copy(ihi_hbm.at[pl.ds(base, bpw)], ihi_v)
        pltpu.sync_copy(ilo_hbm.at[pl.ds(base, bpw)], ilo_v)
        e_lo = lax.iota(jnp.int32, LANES)
        e_hi = e_lo + LANES

        def gather_table(hi_v, lo_v, tab_hbm, rows_v):
            def group(g, _):
                hvec = hi_v[pl.ds(g * NBUF, NBUF)]
                lvec = lo_v[pl.ds(g * NBUF, NBUF)]
                for j in range(NBUF):
                    pltpu.async_copy(
                        tab_hbm.at[:, pl.ds(hvec[j] * 128, 128)],
                        ring_v.at[j], sems[j])
                for j in range(NBUF):
                    pltpu.make_async_copy(
                        tab_hbm.at[:, pl.ds(0, 128)],
                        ring_v.at[j], sems[j]).wait()
                    lane = jnp.full((LANES,), lvec[j], jnp.int32)
                    k = jnp.full((LANES,), g * NBUF + j, jnp.int32)
                    vlo = plsc.load_gather(ring_v.at[j], [e_lo, lane])
                    vhi = plsc.load_gather(ring_v.at[j], [e_hi, lane])
                    plsc.store_scatter(rows_v, [e_lo, k], vlo)
                    plsc.store_scatter(rows_v, [e_hi, k], vhi)
                return ()

            lax.fori_loop(0, bpw // NBUF, group, ())

        gather_table(uhi_v, ulo_v, utab_hbm, urows_v)
        gather_table(ihi_v, ilo_v, itab_hbm, irows_v)
        pltpu.sync_copy(urows_v, uout_hbm.at[:, pl.ds(base, bpw)])
        pltpu.sync_copy(irows_v, iout_hbm.at[:, pl.ds(base, bpw)])

    return sc_gather


def _mlp_body(u_ref, i_ref, w1u_ref, w1i_ref, b1_ref, w2_ref, b2_ref,
              w3_ref, b3_ref, o_ref):
    h1 = jnp.dot(w1u_ref[...], u_ref[...], preferred_element_type=jnp.float32)
    h1 = h1 + jnp.dot(w1i_ref[...], i_ref[...],
                      preferred_element_type=jnp.float32)
    h1 = jnp.maximum(h1 + b1_ref[...], 0.0)
    h2 = jnp.dot(w2_ref[...], h1, preferred_element_type=jnp.float32)
    h2 = jnp.maximum(h2 + b2_ref[...], 0.0)
    z = jnp.dot(w3_ref[...], h2, preferred_element_type=jnp.float32)
    o_ref[...] = jax.nn.sigmoid(z + b3_ref[...])


def kernel(user_input, item_input, user_table, item_table,
           W1, b1, W2, b2, W3, b3):
    batch = user_input.shape[0]
    uidx = user_input.astype(jnp.int32)
    iidx = item_input.astype(jnp.int32)
    uhi, ulo = uidx >> 7, uidx & 127
    ihi, ilo = iidx >> 7, iidx & 127

    u_t, i_t = _sc_gather_make(batch)(
        uhi, ulo, ihi, ilo, user_table.T, item_table.T)

    bm = 2048
    pred_t = pl.pallas_call(
        _mlp_body,
        grid=(batch // bm,),
        in_specs=[
            pl.BlockSpec((EMB, bm), lambda b: (0, b)),
            pl.BlockSpec((EMB, bm), lambda b: (0, b)),
            pl.BlockSpec((64, EMB), lambda b: (0, 0)),
            pl.BlockSpec((64, EMB), lambda b: (0, 0)),
            pl.BlockSpec((64, 1), lambda b: (0, 0)),
            pl.BlockSpec((EMB, 64), lambda b: (0, 0)),
            pl.BlockSpec((EMB, 1), lambda b: (0, 0)),
            pl.BlockSpec((1, EMB), lambda b: (0, 0)),
            pl.BlockSpec((1, 1), lambda b: (0, 0)),
        ],
        out_specs=pl.BlockSpec((1, bm), lambda b: (0, b)),
        out_shape=jax.ShapeDtypeStruct((1, batch), jnp.float32),
    )(u_t, i_t, W1[:EMB].T, W1[EMB:].T, b1.reshape(64, 1),
      W2.T, b2.reshape(EMB, 1), W3.T, b3.reshape(1, 1))
    return pred_t.reshape(batch, 1)


# speedup vs baseline: 6.0342x; 1.1530x over previous
import functools

import jax
import jax.numpy as jnp
from jax import lax
from jax.experimental import pallas as pl
from jax.experimental.pallas import tpu as pltpu
from jax.experimental.pallas import tpu_sc as plsc

EMB = 32
NC, NS = 2, 16
NW = NC * NS
LANES = 16
NBUF = 16


def _sc_gather_make(batch):
    bpw = batch // NW

    @functools.partial(
        pl.kernel,
        out_type=(
            jax.ShapeDtypeStruct((EMB, batch), jnp.float32),
            jax.ShapeDtypeStruct((EMB, batch), jnp.float32),
        ),
        mesh=plsc.VectorSubcoreMesh(core_axis_name="c", subcore_axis_name="s"),
        scratch_types=[
            pltpu.VMEM((bpw,), jnp.int32),
            pltpu.VMEM((bpw,), jnp.int32),
            pltpu.VMEM((bpw,), jnp.int32),
            pltpu.VMEM((bpw,), jnp.int32),
            pltpu.VMEM((NBUF, EMB, 128), jnp.float32),
            pltpu.VMEM((EMB, bpw), jnp.float32),
            pltpu.VMEM((EMB, bpw), jnp.float32),
        ] + [pltpu.SemaphoreType.DMA] * NBUF,
        compiler_params=pltpu.CompilerParams(needs_layout_passes=False),
    )
    def sc_gather(uhi_hbm, ulo_hbm, ihi_hbm, ilo_hbm, utab_hbm, itab_hbm,
                  uout_hbm, iout_hbm, uhi_v, ulo_v, ihi_v, ilo_v,
                  ring_v, urows_v, irows_v, *sems):
        wid = lax.axis_index("s") * NC + lax.axis_index("c")
        base = wid * bpw
        pltpu.sync_copy(uhi_hbm.at[pl.ds(base, bpw)], uhi_v)
        pltpu.sync_copy(ulo_hbm.at[pl.ds(base, bpw)], ulo_v)
        pltpu.sync_copy(ihi_hbm.at[pl.ds(base, bpw)], ihi_v)
        pltpu.sync_copy(ilo_hbm.at[pl.ds(base, bpw)], ilo_v)
        e_lo = lax.iota(jnp.int32, LANES)
        e_hi = e_lo + LANES

        def gather_table(hi_v, lo_v, tab_hbm, rows_v):
            ngrp = bpw // NBUF

            def fire_group(g):
                hvec = hi_v[pl.ds(g * NBUF, NBUF)]
                for j in range(NBUF):
                    pltpu.async_copy(
                        tab_hbm.at[:, pl.ds(hvec[j] * 128, 128)],
                        ring_v.at[j], sems[j])

            def drain_extract_fire(g, refire):
                lvec = lo_v[pl.ds(g * NBUF, NBUF)]
                hnext = hi_v[pl.ds((g + 1) * NBUF if refire else 0, NBUF)]
                for j in range(NBUF):
                    pltpu.make_async_copy(
                        tab_hbm.at[:, pl.ds(0, 128)],
                        ring_v.at[j], sems[j]).wait()
                    lane = jnp.full((LANES,), lvec[j], jnp.int32)
                    k = jnp.full((LANES,), g * NBUF + j, jnp.int32)
                    vlo = plsc.load_gather(ring_v.at[j], [e_lo, lane])
                    vhi = plsc.load_gather(ring_v.at[j], [e_hi, lane])
                    plsc.store_scatter(rows_v, [e_lo, k], vlo)
                    plsc.store_scatter(rows_v, [e_hi, k], vhi)
                    if refire:
                        pltpu.async_copy(
                            tab_hbm.at[:, pl.ds(hnext[j] * 128, 128)],
                            ring_v.at[j], sems[j])
                return ()

            fire_group(0)
            lax.fori_loop(0, ngrp - 1,
                          lambda g, c: drain_extract_fire(g, True), ())
            drain_extract_fire(ngrp - 1, False)

        gather_table(uhi_v, ulo_v, utab_hbm, urows_v)
        gather_table(ihi_v, ilo_v, itab_hbm, irows_v)
        pltpu.sync_copy(urows_v, uout_hbm.at[:, pl.ds(base, bpw)])
        pltpu.sync_copy(irows_v, iout_hbm.at[:, pl.ds(base, bpw)])

    return sc_gather


def _mlp_body(u_ref, i_ref, w1u_ref, w1i_ref, b1_ref, w2_ref, b2_ref,
              w3_ref, b3_ref, o_ref):
    h1 = jnp.dot(w1u_ref[...], u_ref[...], preferred_element_type=jnp.float32)
    h1 = h1 + jnp.dot(w1i_ref[...], i_ref[...],
                      preferred_element_type=jnp.float32)
    h1 = jnp.maximum(h1 + b1_ref[...], 0.0)
    h2 = jnp.dot(w2_ref[...], h1, preferred_element_type=jnp.float32)
    h2 = jnp.maximum(h2 + b2_ref[...], 0.0)
    z = jnp.dot(w3_ref[...], h2, preferred_element_type=jnp.float32)
    o_ref[...] = jax.nn.sigmoid(z + b3_ref[...])


def kernel(user_input, item_input, user_table, item_table,
           W1, b1, W2, b2, W3, b3):
    batch = user_input.shape[0]
    uidx = user_input.astype(jnp.int32)
    iidx = item_input.astype(jnp.int32)
    uhi, ulo = uidx >> 7, uidx & 127
    ihi, ilo = iidx >> 7, iidx & 127

    u_t, i_t = _sc_gather_make(batch)(
        uhi, ulo, ihi, ilo, user_table.T, item_table.T)

    bm = 2048
    pred_t = pl.pallas_call(
        _mlp_body,
        grid=(batch // bm,),
        in_specs=[
            pl.BlockSpec((EMB, bm), lambda b: (0, b)),
            pl.BlockSpec((EMB, bm), lambda b: (0, b)),
            pl.BlockSpec((64, EMB), lambda b: (0, 0)),
            pl.BlockSpec((64, EMB), lambda b: (0, 0)),
            pl.BlockSpec((64, 1), lambda b: (0, 0)),
            pl.BlockSpec((EMB, 64), lambda b: (0, 0)),
            pl.BlockSpec((EMB, 1), lambda b: (0, 0)),
            pl.BlockSpec((1, EMB), lambda b: (0, 0)),
            pl.BlockSpec((1, 1), lambda b: (0, 0)),
        ],
        out_specs=pl.BlockSpec((1, bm), lambda b: (0, b)),
        out_shape=jax.ShapeDtypeStruct((1, batch), jnp.float32),
    )(u_t, i_t, W1[:EMB].T, W1[EMB:].T, b1.reshape(64, 1),
      W2.T, b2.reshape(EMB, 1), W3.T, b3.reshape(1, 1))
    return pred_t.reshape(batch, 1)


# speedup vs baseline: 6.0873x; 1.0088x over previous
import functools

import jax
import jax.numpy as jnp
from jax import lax
from jax.experimental import pallas as pl
from jax.experimental.pallas import tpu as pltpu
from jax.experimental.pallas import tpu_sc as plsc

EMB = 32
NC, NS = 2, 16
NW = NC * NS
LANES = 16
NBUF = 16


def _sc_gather_make(batch):
    bpw = batch // NW

    @functools.partial(
        pl.kernel,
        out_type=(
            jax.ShapeDtypeStruct((EMB, batch), jnp.float32),
            jax.ShapeDtypeStruct((EMB, batch), jnp.float32),
        ),
        mesh=plsc.VectorSubcoreMesh(core_axis_name="c", subcore_axis_name="s"),
        scratch_types=[
            pltpu.VMEM((bpw,), jnp.int32),
            pltpu.VMEM((bpw,), jnp.int32),
            pltpu.VMEM((bpw,), jnp.int32),
            pltpu.VMEM((bpw,), jnp.int32),
            pltpu.VMEM((NBUF, EMB, 128), jnp.float32),
            pltpu.VMEM((EMB, bpw), jnp.float32),
            pltpu.VMEM((EMB, bpw), jnp.float32),
        ] + [pltpu.SemaphoreType.DMA] * NBUF,
        compiler_params=pltpu.CompilerParams(needs_layout_passes=False),
    )
    def sc_gather(uhi_hbm, ulo_hbm, ihi_hbm, ilo_hbm, utab_hbm, itab_hbm,
                  uout_hbm, iout_hbm, uhi_v, ulo_v, ihi_v, ilo_v,
                  ring_v, urows_v, irows_v, *sems):
        wid = lax.axis_index("s") * NC + lax.axis_index("c")
        base = wid * bpw
        idx_copies = [
            pltpu.async_copy(uhi_hbm.at[pl.ds(base, bpw)], uhi_v, sems[0]),
            pltpu.async_copy(ulo_hbm.at[pl.ds(base, bpw)], ulo_v, sems[1]),
            pltpu.async_copy(ihi_hbm.at[pl.ds(base, bpw)], ihi_v, sems[2]),
            pltpu.async_copy(ilo_hbm.at[pl.ds(base, bpw)], ilo_v, sems[3]),
        ]
        for cp in idx_copies:
            cp.wait()
        e_lo = lax.iota(jnp.int32, LANES)
        e_hi = e_lo + LANES

        def gather_table(hi_v, lo_v, tab_hbm, rows_v):
            ngrp = bpw // NBUF

            def fire_group(g):
                hvec = hi_v[pl.ds(g * NBUF, NBUF)]
                for j in range(NBUF):
                    pltpu.async_copy(
                        tab_hbm.at[:, pl.ds(hvec[j] * 128, 128)],
                        ring_v.at[j], sems[j])

            def drain_extract_fire(g, refire):
                lvec = lo_v[pl.ds(g * NBUF, NBUF)]
                hnext = hi_v[pl.ds((g + 1) * NBUF if refire else 0, NBUF)]
                for j in range(NBUF):
                    pltpu.make_async_copy(
                        tab_hbm.at[:, pl.ds(0, 128)],
                        ring_v.at[j], sems[j]).wait()
                    lane = jnp.full((LANES,), lvec[j], jnp.int32)
                    k = jnp.full((LANES,), g * NBUF + j, jnp.int32)
                    vlo = plsc.load_gather(ring_v.at[j], [e_lo, lane])
                    vhi = plsc.load_gather(ring_v.at[j], [e_hi, lane])
                    plsc.store_scatter(rows_v, [e_lo, k], vlo)
                    plsc.store_scatter(rows_v, [e_hi, k], vhi)
                    if refire:
                        pltpu.async_copy(
                            tab_hbm.at[:, pl.ds(hnext[j] * 128, 128)],
                            ring_v.at[j], sems[j])
                return ()

            fire_group(0)
            lax.fori_loop(0, ngrp - 1,
                          lambda g, c: drain_extract_fire(g, True), ())
            drain_extract_fire(ngrp - 1, False)

        gather_table(uhi_v, ulo_v, utab_hbm, urows_v)
        gather_table(ihi_v, ilo_v, itab_hbm, irows_v)
        pltpu.sync_copy(urows_v, uout_hbm.at[:, pl.ds(base, bpw)])
        pltpu.sync_copy(irows_v, iout_hbm.at[:, pl.ds(base, bpw)])

    return sc_gather


def _mlp_body(u_ref, i_ref, w1u_ref, w1i_ref, b1_ref, w2_ref, b2_ref,
              w3_ref, b3_ref, o_ref):
    h1 = jnp.dot(w1u_ref[...], u_ref[...], preferred_element_type=jnp.float32)
    h1 = h1 + jnp.dot(w1i_ref[...], i_ref[...],
                      preferred_element_type=jnp.float32)
    h1 = jnp.maximum(h1 + b1_ref[...], 0.0)
    h2 = jnp.dot(w2_ref[...], h1, preferred_element_type=jnp.float32)
    h2 = jnp.maximum(h2 + b2_ref[...], 0.0)
    z = jnp.dot(w3_ref[...], h2, preferred_element_type=jnp.float32)
    o_ref[...] = jax.nn.sigmoid(z + b3_ref[...])


def kernel(user_input, item_input, user_table, item_table,
           W1, b1, W2, b2, W3, b3):
    batch = user_input.shape[0]
    uidx = user_input.astype(jnp.int32)
    iidx = item_input.astype(jnp.int32)
    uhi, ulo = uidx >> 7, uidx & 127
    ihi, ilo = iidx >> 7, iidx & 127

    u_t, i_t = _sc_gather_make(batch)(
        uhi, ulo, ihi, ilo, user_table.T, item_table.T)

    bm = 4096
    pred_t = pl.pallas_call(
        _mlp_body,
        grid=(batch // bm,),
        in_specs=[
            pl.BlockSpec((EMB, bm), lambda b: (0, b)),
            pl.BlockSpec((EMB, bm), lambda b: (0, b)),
            pl.BlockSpec((64, EMB), lambda b: (0, 0)),
            pl.BlockSpec((64, EMB), lambda b: (0, 0)),
            pl.BlockSpec((64, 1), lambda b: (0, 0)),
            pl.BlockSpec((EMB, 64), lambda b: (0, 0)),
            pl.BlockSpec((EMB, 1), lambda b: (0, 0)),
            pl.BlockSpec((1, EMB), lambda b: (0, 0)),
            pl.BlockSpec((1, 1), lambda b: (0, 0)),
        ],
        out_specs=pl.BlockSpec((1, bm), lambda b: (0, b)),
        out_shape=jax.ShapeDtypeStruct((1, batch), jnp.float32),
    )(u_t, i_t, W1[:EMB].T, W1[EMB:].T, b1.reshape(64, 1),
      W2.T, b2.reshape(EMB, 1), W3.T, b3.reshape(1, 1))
    return pred_t.reshape(batch, 1)
